# baseline (speedup 1.0000x reference)
import jax
import jax.numpy as jnp
from jax import lax
from jax.experimental import pallas as pl
from jax.experimental.pallas import tpu as pltpu
from jax.experimental.pallas import tpu_sc as plsc

B, S, D = 4, 8192, 768
NC, NS = 2, 16
NW = NC * NS
SEQ_PER_W = S // NW
CHUNK_ROWS = 16
CHUNK = CHUNK_ROWS * D
N_PE = SEQ_PER_W // CHUNK_ROWS
N_PH = N_PE * B
VECS = CHUNK // 16
NXB = 8
LOOK = 4


def _sc_add(x_hbm, pe_hbm, o_hbm, *scratch):
    x_bufs = scratch[:NXB]
    pe_bufs = scratch[NXB:NXB + 2]
    x_sems = scratch[NXB + 2:2 * NXB + 2]
    pe_sems = scratch[2 * NXB + 2:2 * NXB + 4]
    st_sems = scratch[2 * NXB + 4:3 * NXB + 4]

    wid = lax.axis_index("s") * NC + lax.axis_index("c")
    seq0 = wid * SEQ_PER_W

    def rows(c):
        p, b = divmod(c, B)
        return b, seq0 + p * CHUNK_ROWS

    def start_x(c, xb):
        b, r0 = rows(c)
        pltpu.make_async_copy(
            x_hbm.at[b, pl.ds(r0, CHUNK_ROWS), :], x_bufs[xb],
            x_sems[xb]).start()

    def start_pe(p, par):
        r0 = seq0 + p * CHUNK_ROWS
        pltpu.make_async_copy(
            pe_hbm.at[pl.ds(r0, CHUNK_ROWS), :], pe_bufs[par],
            pe_sems[par]).start()

    def start_store(c, xb):
        b, r0 = rows(c)
        pltpu.make_async_copy(
            x_bufs[xb], o_hbm.at[b, pl.ds(r0, CHUNK_ROWS), :],
            st_sems[xb]).start()

    def drain(sem, buf):
        pltpu.make_async_copy(
            x_hbm.at[0, pl.ds(0, CHUNK_ROWS), :], buf, sem).wait()

    start_pe(0, 0)
    start_pe(1, 1)
    for c in range(LOOK):
        start_x(c, c % NXB)

    GP = 2 * B
    NG = N_PH // GP

    def group(g, carry):
        for j in range(GP):
            c = g * GP + j
            par = j // B
            xb = j % NXB
            if j % B == 0:
                drain(pe_sems[par], pe_bufs[par])
            drain(x_sems[xb], x_bufs[xb])

            def add_body(k, c2, _xv=x_bufs[xb], _pv=pe_bufs[par]):
                off = k * 16
                for r in range(CHUNK_ROWS):
                    plsc.addupdate(_xv.at[r, pl.ds(off, 16)],
                                   _pv[r, pl.ds(off, 16)])
                return c2

            lax.fori_loop(0, D // 16, add_body, 0)

            start_store(c, xb)
            nxb = (j + LOOK) % NXB
            if j < LOOK:

                @pl.when(g > 0)
                def _():
                    drain(st_sems[nxb], x_bufs[nxb])
                    start_x(c + LOOK, nxb)

                @pl.when(g == 0)
                def _():
                    start_x(c + LOOK, nxb)
            elif j >= GP - LOOK:

                @pl.when(g < NG - 1)
                def _():
                    drain(st_sems[nxb], x_bufs[nxb])
                    start_x(c + LOOK, nxb)
            else:
                drain(st_sems[nxb], x_bufs[nxb])
                start_x(c + LOOK, nxb)
            if j % B == B - 1:

                @pl.when(g < NG - 1)
                def _(_par=par):
                    start_pe(2 * (g + 1) + _par, _par)
        return carry

    lax.fori_loop(0, NG, group, 0)

    for c in range(N_PH - NXB, N_PH):
        drain(st_sems[c % NXB], x_bufs[c % NXB])


def kernel(x, pe_table):
    mesh = plsc.VectorSubcoreMesh(core_axis_name="c", subcore_axis_name="s")
    run = pl.kernel(
        _sc_add,
        mesh=mesh,
        out_type=jax.ShapeDtypeStruct((B, S, D), jnp.float32),
        scratch_types=(
            [pltpu.VMEM((CHUNK_ROWS, D), jnp.float32) for _ in range(NXB + 2)]
            + [pltpu.SemaphoreType.DMA for _ in range(NXB + 2 + NXB)]
        ),
    )
    return run(x, pe_table)


# speedup vs baseline: 1.0699x; 1.0699x over previous
import jax
import jax.numpy as jnp
from jax import lax
from jax.experimental import pallas as pl
from jax.experimental.pallas import tpu as pltpu
from jax.experimental.pallas import tpu_sc as plsc

B, S, D = 4, 8192, 768
NC, NS = 2, 16
NW = NC * NS
SEQ_PER_W = S // NW
CHUNK_ROWS = 16
CHUNK = CHUNK_ROWS * D
N_PE = SEQ_PER_W // CHUNK_ROWS
N_PH = N_PE * B
VECS = CHUNK // 16
NXB = 8
LOOK = 4


def _sc_add(x_hbm, pe_hbm, o_hbm, *scratch):
    x_bufs = scratch[:NXB]
    pe_bufs = scratch[NXB:NXB + 2]
    x_sems = scratch[NXB + 2:2 * NXB + 2]
    pe_sems = scratch[2 * NXB + 2:2 * NXB + 4]
    st_sems = scratch[2 * NXB + 4:3 * NXB + 4]

    wid = lax.axis_index("s") * NC + lax.axis_index("c")
    seq0 = wid * SEQ_PER_W

    def rows(c):
        p, b = divmod(c, B)
        return b, seq0 + p * CHUNK_ROWS

    def start_x(c, xb):
        b, r0 = rows(c)
        buf, sem = x_bufs[xb], x_sems[xb]

        def row(r, carry):
            pltpu.make_async_copy(
                x_hbm.at[b, r0 + r], buf.at[pl.ds(r * D, D)], sem).start()
            return carry
        lax.fori_loop(0, CHUNK_ROWS, row, 0)

    def start_pe(p, par):
        r0 = seq0 + p * CHUNK_ROWS
        buf, sem = pe_bufs[par], pe_sems[par]

        def row(r, carry):
            pltpu.make_async_copy(
                pe_hbm.at[r0 + r], buf.at[pl.ds(r * D, D)], sem).start()
            return carry
        lax.fori_loop(0, CHUNK_ROWS, row, 0)

    def start_store(c, xb):
        b, r0 = rows(c)
        buf, sem = x_bufs[xb], st_sems[xb]

        def row(r, carry):
            pltpu.make_async_copy(
                buf.at[pl.ds(r * D, D)], o_hbm.at[b, r0 + r], sem).start()
            return carry
        lax.fori_loop(0, CHUNK_ROWS, row, 0)

    def drain(sem, buf):
        def row(r, carry):
            pltpu.make_async_copy(
                x_hbm.at[0, 0], buf.at[pl.ds(0, D)], sem).wait()
            return carry
        lax.fori_loop(0, CHUNK_ROWS, row, 0)

    start_pe(0, 0)
    start_pe(1, 1)
    for c in range(LOOK):
        start_x(c, c % NXB)

    GP = 2 * B
    NG = N_PH // GP

    def group(g, carry):
        for j in range(GP):
            c = g * GP + j
            par = j // B
            xb = j % NXB
            if j % B == 0:
                drain(pe_sems[par], pe_bufs[par])
            drain(x_sems[xb], x_bufs[xb])

            def add_body(k, c2, _xv=x_bufs[xb], _pv=pe_bufs[par]):
                off = k * 16
                plsc.addupdate(_xv.at[pl.ds(off, 16)], _pv[pl.ds(off, 16)])
                return c2

            lax.fori_loop(0, VECS, add_body, 0, unroll=16)

            start_store(c, xb)
            nxb = (j + LOOK) % NXB
            if j < LOOK:

                @pl.when(g > 0)
                def _():
                    drain(st_sems[nxb], x_bufs[nxb])
                    start_x(c + LOOK, nxb)

                @pl.when(g == 0)
                def _():
                    start_x(c + LOOK, nxb)
            elif j >= GP - LOOK:

                @pl.when(g < NG - 1)
                def _():
                    drain(st_sems[nxb], x_bufs[nxb])
                    start_x(c + LOOK, nxb)
            else:
                drain(st_sems[nxb], x_bufs[nxb])
                start_x(c + LOOK, nxb)
            if j % B == B - 1:

                @pl.when(g < NG - 1)
                def _(_par=par):
                    start_pe(2 * (g + 1) + _par, _par)
        return carry

    lax.fori_loop(0, NG, group, 0)

    for c in range(N_PH - NXB, N_PH):
        drain(st_sems[c % NXB], x_bufs[c % NXB])


def kernel(x, pe_table):
    mesh = plsc.VectorSubcoreMesh(core_axis_name="c", subcore_axis_name="s")
    run = pl.kernel(
        _sc_add,
        mesh=mesh,
        out_type=jax.ShapeDtypeStruct((B, S, D), jnp.float32),
        scratch_types=(
            [pltpu.VMEM((CHUNK,), jnp.float32) for _ in range(NXB + 2)]
            + [pltpu.SemaphoreType.DMA for _ in range(NXB + 2 + NXB)]
        ),
    )
    return run(x, pe_table)


# speedup vs baseline: 1.0742x; 1.0040x over previous
import jax
import jax.numpy as jnp
from jax import lax
from jax.experimental import pallas as pl
from jax.experimental.pallas import tpu as pltpu
from jax.experimental.pallas import tpu_sc as plsc

B, S, D = 4, 8192, 768
NC, NS = 2, 16
NW = NC * NS
SEQ_PER_W = S // NW
CHUNK_ROWS = 16
CHUNK = CHUNK_ROWS * D
N_PE = SEQ_PER_W // CHUNK_ROWS
N_PH = N_PE * B
VECS = CHUNK // 16
NXB = 8
LOOK = 4


def _sc_add(x_hbm, pe_hbm, o_hbm, *scratch):
    x_bufs = scratch[:NXB]
    pe_bufs = scratch[NXB:NXB + 2]
    x_sems = scratch[NXB + 2:2 * NXB + 2]
    pe_sems = scratch[2 * NXB + 2:2 * NXB + 4]
    st_sems = scratch[2 * NXB + 4:3 * NXB + 4]

    wid = lax.axis_index("s") * NC + lax.axis_index("c")
    seq0 = wid * SEQ_PER_W

    def rows(c):
        p, b = divmod(c, B)
        return b, seq0 + p * CHUNK_ROWS

    def start_x(c, xb):
        b, r0 = rows(c)
        buf, sem = x_bufs[xb], x_sems[xb]

        def row(r, carry):
            pltpu.make_async_copy(
                x_hbm.at[b, r0 + r], buf.at[pl.ds(r * D, D)], sem).start()
            return carry
        lax.fori_loop(0, CHUNK_ROWS, row, 0)

    def start_pe(p, par):
        r0 = seq0 + p * CHUNK_ROWS
        buf, sem = pe_bufs[par], pe_sems[par]

        def row(r, carry):
            pltpu.make_async_copy(
                pe_hbm.at[r0 + r], buf.at[pl.ds(r * D, D)], sem).start()
            return carry
        lax.fori_loop(0, CHUNK_ROWS, row, 0)

    def start_store(c, xb):
        b, r0 = rows(c)
        buf, sem = x_bufs[xb], st_sems[xb]

        def row(r, carry):
            pltpu.make_async_copy(
                buf.at[pl.ds(r * D, D)], o_hbm.at[b, r0 + r], sem).start()
            return carry
        lax.fori_loop(0, CHUNK_ROWS, row, 0)

    def drain(sem, buf):
        def row(r, carry):
            pltpu.make_async_copy(
                x_hbm.at[0, 0], buf.at[pl.ds(0, D)], sem).wait()
            return carry
        lax.fori_loop(0, CHUNK_ROWS, row, 0)

    start_pe(0, 0)
    start_pe(1, 1)
    for c in range(LOOK):
        start_x(c, c % NXB)

    GP = 2 * B
    NG = N_PH // GP

    def group(g, carry):
        for j in range(GP):
            c = g * GP + j
            par = j // B
            xb = j % NXB
            if j % B == 0:
                drain(pe_sems[par], pe_bufs[par])
            drain(x_sems[xb], x_bufs[xb])

            def add_body(k, c2, _xv=x_bufs[xb], _pv=pe_bufs[par]):
                off = k * 16
                plsc.addupdate(_xv.at[pl.ds(off, 16)], _pv[pl.ds(off, 16)])
                return c2

            lax.fori_loop(0, VECS, add_body, 0, unroll=8)

            start_store(c, xb)
            nxb = (j + LOOK) % NXB
            if j < LOOK:

                @pl.when(g > 0)
                def _():
                    drain(st_sems[nxb], x_bufs[nxb])
                    start_x(c + LOOK, nxb)

                @pl.when(g == 0)
                def _():
                    start_x(c + LOOK, nxb)
            elif j >= GP - LOOK:

                @pl.when(g < NG - 1)
                def _():
                    drain(st_sems[nxb], x_bufs[nxb])
                    start_x(c + LOOK, nxb)
            else:
                drain(st_sems[nxb], x_bufs[nxb])
                start_x(c + LOOK, nxb)
            if j % B == B - 1:

                @pl.when(g < NG - 1)
                def _(_par=par):
                    start_pe(2 * (g + 1) + _par, _par)
        return carry

    lax.fori_loop(0, NG, group, 0)

    for c in range(N_PH - NXB, N_PH):
        drain(st_sems[c % NXB], x_bufs[c % NXB])


def kernel(x, pe_table):
    mesh = plsc.VectorSubcoreMesh(core_axis_name="c", subcore_axis_name="s")
    run = pl.kernel(
        _sc_add,
        mesh=mesh,
        out_type=jax.ShapeDtypeStruct((B, S, D), jnp.float32),
        scratch_types=(
            [pltpu.VMEM((CHUNK,), jnp.float32) for _ in range(NXB + 2)]
            + [pltpu.SemaphoreType.DMA for _ in range(NXB + 2 + NXB)]
        ),
    )
    return run(x, pe_table)
